# baseline (speedup 1.0000x reference)
import dataclasses
import functools

import jax
import jax.numpy as jnp
from jax import lax
from jax.experimental import pallas as pl
from jax.experimental.pallas import tpu as pltpu
from jax.experimental.pallas import tpu_sc as plsc

_B, _NN, _LB, _TRAJ = 64, 10000, 8, 64
_NP = 10240
_RPB = _NP // 128


def _tc_a(x, fc1_W, fc1_b, emb_stop, Wl, Wr):
    def body(fb_ref, x_ref, fw_ref, emb_ref, wl_ref, wr_ref,
             o_ref, phi_ref):
        w = fw_ref[0, :]
        fb = fb_ref[0]
        for r in range(4):
            row = jnp.sum(x_ref[r] * w[:, None], axis=0) + fb
            o_ref[pl.ds(r * _NP, _NN)] = row

        @pl.when(pl.program_id(0) == 0)
        def _():
            embT = emb_ref[...]
            phi_ref[pl.ds(0, _NN)] = jnp.sum(
                embT * wl_ref[0, 36:48][:, None], axis=0)
            phi_ref[pl.ds(_NP, _NN)] = jnp.sum(
                embT * wr_ref[0, 36:48][:, None], axis=0)

    return pl.pallas_call(
        body,
        grid=(16,),
        in_specs=[
            pl.BlockSpec(memory_space=pltpu.SMEM),
            pl.BlockSpec((4, _LB, _NN), lambda i: (i, 0, 0)),
            pl.BlockSpec((1, _LB), lambda i: (0, 0)),
            pl.BlockSpec((12, _NN), lambda i: (0, 0)),
            pl.BlockSpec((1, 50), lambda i: (0, 0)),
            pl.BlockSpec((1, 50), lambda i: (0, 0)),
        ],
        out_specs=[
            pl.BlockSpec((4 * _NP,), lambda i: (i,)),
            pl.BlockSpec((2 * _NP,), lambda i: (0,)),
        ],
        out_shape=[
            jax.ShapeDtypeStruct((_B * _NP,), jnp.float32),
            jax.ShapeDtypeStruct((2 * _NP,), jnp.float32),
        ],
    )(fc1_b, x, fc1_W, emb_stop.T, Wl, Wr)


def _sc_kernel(stops, out_flat, phi_flat, x_dist, pack, x_week, x_feat_flat,
               emb_week_flat):
    mesh = plsc.VectorSubcoreMesh(core_axis_name="c", subcore_axis_name="s")
    cp = pltpu.CompilerParams(use_tc_tiling_on_sc=False)
    if "needs_layout_passes" in pltpu.CompilerParams.__dataclass_fields__:
        cp = dataclasses.replace(cp, needs_layout_passes=False)

    @functools.partial(
        pl.kernel,
        mesh=mesh,
        out_type=jax.ShapeDtypeStruct((_B * _NP,), jnp.float32),
        scratch_types=[
            pltpu.VMEM((_TRAJ,), jnp.int32),
            pltpu.VMEM((_TRAJ,), jnp.int32),
            pltpu.VMEM((_TRAJ,), jnp.int32),
            pltpu.VMEM((_TRAJ,), jnp.float32),
            pltpu.VMEM((_TRAJ,), jnp.float32),
            pltpu.VMEM((_TRAJ,), jnp.float32),
            pltpu.VMEM((_TRAJ,), jnp.float32),
            pltpu.VMEM((_NN,), jnp.float32),
            pltpu.VMEM((_NN,), jnp.float32),
            pltpu.VMEM((_NN,), jnp.float32),
            pltpu.VMEM((_NN,), jnp.float32),
            pltpu.VMEM((144,), jnp.float32),
            pltpu.VMEM((64,), jnp.int32),
            pltpu.VMEM((128,), jnp.float32),
            pltpu.VMEM((240,), jnp.float32),
            pltpu.SemaphoreType.DMA,
            pltpu.SemaphoreType.DMA,
        ],
        compiler_params=cp,
    )
    def sck(stops_hbm, oflat_hbm, phi_hbm, xd_hbm, pack_hbm, xw_hbm, xf_hbm,
            ew_hbm, sage_hbm,
            sb, oidx, pidx, osrc_v, phl_v, phr_v, xd_v,
            aggl0, cntl0, aggl1, cntl1, packb, xwbuf, xfbuf, ewbuf,
            sem_g, sem_out):
        wid = lax.axis_index("s") * 2 + lax.axis_index("c")
        pltpu.sync_copy(pack_hbm, packb)
        pltpu.sync_copy(xw_hbm, xwbuf)
        pltpu.sync_copy(xf_hbm, xfbuf)
        pltpu.sync_copy(ew_hbm, ewbuf)
        iota = lax.iota(jnp.int32, 16)
        p0 = packb[pl.ds(0, 16)]
        wl48 = jnp.sum(jnp.where(iota == 0, p0, 0.0))
        wl49 = jnp.sum(jnp.where(iota == 1, p0, 0.0))
        blv = jnp.sum(jnp.where(iota == 2, p0, 0.0))
        ones16 = jnp.ones((16,), jnp.float32)
        masks = [(iota == j) for j in range(16)]
        wlA = packb[pl.ds(16, 16)]
        wlB = packb[pl.ds(32, 16)]
        wlC = packb[pl.ds(48, 16)]
        wrA = packb[pl.ds(80, 16)]
        wrB = packb[pl.ds(96, 16)]
        wrC = packb[pl.ds(112, 16)]

        out_copies = []
        for r, (aggl, cntl) in enumerate(((aggl0, cntl0), (aggl1, cntl1))):
            b = wid * 2 + r
            bv = jnp.full((16,), 0, jnp.int32) + b
            pltpu.sync_copy(stops_hbm.at[b], sb)
            boff = b * _NP
            for u in range(4):
                sbu = sb[pl.ds(u * 16, 16)]
                oidx[pl.ds(u * 16, 16)] = sbu + boff
                pidx[pl.ds(u * 16, 16)] = sbu + _NP
            cps = [
                pltpu.async_copy(oflat_hbm.at[oidx], osrc_v, sem_g),
                pltpu.async_copy(phi_hbm.at[sb], phl_v, sem_g),
                pltpu.async_copy(phi_hbm.at[pidx], phr_v, sem_g),
                pltpu.async_copy(xd_hbm.at[sb], xd_v, sem_g),
            ]

            @pl.loop(0, _NN, step=400)
            def _(i):
                z = jnp.zeros((16,), jnp.float32)
                for u in range(25):
                    aggl[pl.ds(i + u * 16, 16)] = z
                    cntl[pl.ds(i + u * 16, 16)] = z

            for c in cps:
                c.wait()

            phl = jnp.zeros((16,), jnp.float32)
            phr = jnp.zeros((16,), jnp.float32)
            for u in range(4):
                phl = phl + phl_v[pl.ds(u * 16, 16)]
                phr = phr + phr_v[pl.ds(u * 16, 16)]
            wv = plsc.load_gather(xwbuf, [bv]) * 34
            ew0 = plsc.load_gather(ewbuf, [wv + iota])
            ew1 = plsc.load_gather(ewbuf, [wv + (iota + 16)])
            ew2 = plsc.load_gather(
                ewbuf, [jnp.minimum(wv + (iota + 32), 237)])
            xfg = plsc.load_gather(
                xfbuf, [jnp.clip(iota + (2 * b - 2), 0, 127)])
            chunk2 = jnp.where(iota < 2, ew2,
                               jnp.where(iota < 4, xfg, 0.0))
            cl = jnp.sum(ew0 * wlA + ew1 * wlB + chunk2 * wlC + phl)
            crbl = blv + jnp.sum(ew0 * wrA + ew1 * wrB + chunk2 * wrC + phr)

            for c in range(4):
                toff = c * 16
                v = (osrc_v[pl.ds(toff, 16)] * wl48
                     + xd_v[pl.ds(toff, 16)] * wl49)
                valid = (iota + toff) < (_TRAJ - 1)
                dstv = plsc.load_gather(
                    sb, [jnp.minimum(iota + (toff + 1), _TRAJ - 1)])
                for j in range(16):
                    m = valid & masks[j]
                    plsc.addupdate_scatter(aggl, [dstv], v, mask=m)
                    plsc.addupdate_scatter(cntl, [dstv], ones16, mask=m)

            @pl.loop(0, _NN, step=80)
            def _(i):
                for u in range(5):
                    sl = pl.ds(i + u * 16, 16)
                    cn = cntl[sl]
                    mean = aggl[sl] / jnp.maximum(cn, 1.0)
                    aggl[sl] = (mean + jnp.where(cn >= 0.5, cl, 0.0)
                                + crbl)

            out_copies.append(pltpu.async_copy(
                aggl, sage_hbm.at[pl.ds(boff, _NN)], sem_out))

        for c in out_copies:
            c.wait()

    return sck(stops, out_flat, phi_flat, x_dist, pack, x_week, x_feat_flat,
               emb_week_flat)


def _tc_b(out2, sage2, xd2, Wr):
    def body(out_ref, sage_ref, xd_ref, wr_ref, o_ref):
        wr = wr_ref[0, :]
        wr48 = wr[48:49]
        wr49 = wr[49:50]
        rowi = lax.broadcasted_iota(jnp.int32, (_RPB, 128), 0)
        lanei = lax.broadcasted_iota(jnp.int32, (_RPB, 128), 1)
        vmask = (rowi * 128 + lanei) < _NN
        xdw = xd_ref[...] * wr49
        gs = []
        for r in range(8):
            sl = pl.ds(r * _RPB, _RPB)
            g = sage_ref[sl, :] + out_ref[sl, :] * wr48 + xdw
            gs.append(jnp.where(vmask, g, -1e30))
        gms = [jnp.max(g) for g in gs]
        es = [jnp.exp(g - gm) for g, gm in zip(gs, gms)]
        lses = [gm + jnp.log(jnp.sum(e)) for gm, e in zip(gms, es)]
        for r in range(8):
            o_ref[pl.ds(r * _RPB, _RPB), :] = gs[r] - lses[r]

    return pl.pallas_call(
        body,
        grid=(8,),
        in_specs=[
            pl.BlockSpec((8 * _RPB, 128), lambda i: (i, 0)),
            pl.BlockSpec((8 * _RPB, 128), lambda i: (i, 0)),
            pl.BlockSpec((_RPB, 128), lambda i: (0, 0)),
            pl.BlockSpec((1, 50), lambda i: (0, 0)),
        ],
        out_specs=pl.BlockSpec((8 * _RPB, 128), lambda i: (i, 0)),
        out_shape=jax.ShapeDtypeStruct((_B * _RPB, 128), jnp.float32),
    )(out2, sage2, xd2, Wr)


def kernel(stops, x, x_dist, x_features, x_week, x_mask, emb_week, emb_stop,
           fc1_W, fc1_b, Wl, bl, Wr):
    f32 = jnp.float32
    stops32 = stops.astype(jnp.int32)
    x_week32 = x_week.astype(jnp.int32)
    z12 = jnp.zeros((12,), f32)
    pack = jnp.concatenate([
        Wl[0, 48:50], bl, jnp.zeros((13,), f32),
        Wl[0, 0:16], Wl[0, 16:32], Wl[0, 32:36], z12, jnp.zeros((16,), f32),
        Wr[0, 0:16], Wr[0, 16:32], Wr[0, 32:36], z12, jnp.zeros((16,), f32),
    ])
    xf_flat = x_features.reshape(-1).astype(f32)
    ew_flat = jnp.concatenate([emb_week.reshape(-1), jnp.zeros((2,), f32)])
    xd2 = jnp.concatenate(
        [x_dist, jnp.zeros((_NP - _NN,), f32)]).reshape(_RPB, 128)

    out_flat, phi_flat = _tc_a(x, fc1_W, fc1_b, emb_stop, Wl, Wr)
    sage_flat = _sc_kernel(stops32, out_flat, phi_flat, x_dist, pack,
                           x_week32, xf_flat, ew_flat)
    logp = _tc_b(out_flat.reshape(_B * _RPB, 128),
                 sage_flat.reshape(_B * _RPB, 128), xd2, Wr)
    logp = logp.reshape(_B, _NP)[:, :_NN]
    return jnp.where(x_mask.astype(bool), jnp.float32(-1e8), logp)


# speedup vs baseline: 1.0739x; 1.0739x over previous
import dataclasses
import functools

import jax
import jax.numpy as jnp
from jax import lax
from jax.experimental import pallas as pl
from jax.experimental.pallas import tpu as pltpu
from jax.experimental.pallas import tpu_sc as plsc

_B, _NN, _LB, _TRAJ = 64, 10000, 8, 64
_NP = 10240
_RPB = _NP // 128


def _tc_a(x, fc1_W, fc1_b, emb_stop, Wl, Wr):
    def body(fb_ref, x_ref, fw_ref, emb_ref, wl_ref, wr_ref,
             o_ref, phi_ref):
        w = fw_ref[0, :]
        fb = fb_ref[0]
        for r in range(8):
            row = jnp.sum(x_ref[r] * w[:, None], axis=0) + fb
            o_ref[pl.ds(r * _NP, _NN)] = row

        @pl.when(pl.program_id(0) == 0)
        def _():
            embT = emb_ref[...]
            phi_ref[pl.ds(0, _NN)] = jnp.sum(
                embT * wl_ref[0, 36:48][:, None], axis=0)
            phi_ref[pl.ds(_NP, _NN)] = jnp.sum(
                embT * wr_ref[0, 36:48][:, None], axis=0)

    return pl.pallas_call(
        body,
        grid=(8,),
        in_specs=[
            pl.BlockSpec(memory_space=pltpu.SMEM),
            pl.BlockSpec((8, _LB, _NN), lambda i: (i, 0, 0)),
            pl.BlockSpec((1, _LB), lambda i: (0, 0)),
            pl.BlockSpec((12, _NN), lambda i: (0, 0)),
            pl.BlockSpec((1, 50), lambda i: (0, 0)),
            pl.BlockSpec((1, 50), lambda i: (0, 0)),
        ],
        out_specs=[
            pl.BlockSpec((8 * _NP,), lambda i: (i,)),
            pl.BlockSpec((2 * _NP,), lambda i: (0,)),
        ],
        out_shape=[
            jax.ShapeDtypeStruct((_B * _NP,), jnp.float32),
            jax.ShapeDtypeStruct((2 * _NP,), jnp.float32),
        ],
    )(fc1_b, x, fc1_W, emb_stop.T, Wl, Wr)


def _sc_kernel(stops, out_flat, phi_flat, x_dist, pack, x_week, x_feat_flat,
               emb_week_flat):
    mesh = plsc.VectorSubcoreMesh(core_axis_name="c", subcore_axis_name="s")
    cp = pltpu.CompilerParams(use_tc_tiling_on_sc=False)
    if "needs_layout_passes" in pltpu.CompilerParams.__dataclass_fields__:
        cp = dataclasses.replace(cp, needs_layout_passes=False)

    @functools.partial(
        pl.kernel,
        mesh=mesh,
        out_type=jax.ShapeDtypeStruct((_B * _NP,), jnp.float32),
        scratch_types=[
            pltpu.VMEM((_TRAJ,), jnp.int32),
            pltpu.VMEM((_TRAJ,), jnp.int32),
            pltpu.VMEM((_TRAJ,), jnp.int32),
            pltpu.VMEM((_TRAJ,), jnp.float32),
            pltpu.VMEM((_TRAJ,), jnp.float32),
            pltpu.VMEM((_TRAJ,), jnp.float32),
            pltpu.VMEM((_TRAJ,), jnp.float32),
            pltpu.VMEM((_NN,), jnp.float32),
            pltpu.VMEM((_NN,), jnp.float32),
            pltpu.VMEM((_NN,), jnp.float32),
            pltpu.VMEM((_NN,), jnp.float32),
            pltpu.VMEM((144,), jnp.float32),
            pltpu.VMEM((64,), jnp.int32),
            pltpu.VMEM((128,), jnp.float32),
            pltpu.VMEM((240,), jnp.float32),
            pltpu.SemaphoreType.DMA,
            pltpu.SemaphoreType.DMA,
        ],
        compiler_params=cp,
    )
    def sck(stops_hbm, oflat_hbm, phi_hbm, xd_hbm, pack_hbm, xw_hbm, xf_hbm,
            ew_hbm, sage_hbm,
            sb, oidx, pidx, osrc_v, phl_v, phr_v, xd_v,
            aggl0, cntl0, aggl1, cntl1, packb, xwbuf, xfbuf, ewbuf,
            sem_g, sem_out):
        wid = lax.axis_index("s") * 2 + lax.axis_index("c")
        pltpu.sync_copy(pack_hbm, packb)
        pltpu.sync_copy(xw_hbm, xwbuf)
        pltpu.sync_copy(xf_hbm, xfbuf)
        pltpu.sync_copy(ew_hbm, ewbuf)
        iota = lax.iota(jnp.int32, 16)
        p0 = packb[pl.ds(0, 16)]
        wl48 = jnp.sum(jnp.where(iota == 0, p0, 0.0))
        wl49 = jnp.sum(jnp.where(iota == 1, p0, 0.0))
        blv = jnp.sum(jnp.where(iota == 2, p0, 0.0))
        ones16 = jnp.ones((16,), jnp.float32)
        masks = [(iota == j) for j in range(16)]
        wlA = packb[pl.ds(16, 16)]
        wlB = packb[pl.ds(32, 16)]
        wlC = packb[pl.ds(48, 16)]
        wrA = packb[pl.ds(80, 16)]
        wrB = packb[pl.ds(96, 16)]
        wrC = packb[pl.ds(112, 16)]

        out_copies = []
        for r, (aggl, cntl) in enumerate(((aggl0, cntl0), (aggl1, cntl1))):
            b = wid * 2 + r
            bv = jnp.full((16,), 0, jnp.int32) + b
            pltpu.sync_copy(stops_hbm.at[b], sb)
            boff = b * _NP
            for u in range(4):
                sbu = sb[pl.ds(u * 16, 16)]
                oidx[pl.ds(u * 16, 16)] = sbu + boff
                pidx[pl.ds(u * 16, 16)] = sbu + _NP
            cps = [
                pltpu.async_copy(oflat_hbm.at[oidx], osrc_v, sem_g),
                pltpu.async_copy(phi_hbm.at[sb], phl_v, sem_g),
                pltpu.async_copy(phi_hbm.at[pidx], phr_v, sem_g),
                pltpu.async_copy(xd_hbm.at[sb], xd_v, sem_g),
            ]

            @pl.loop(0, _NN, step=400)
            def _(i):
                z = jnp.zeros((16,), jnp.float32)
                for u in range(25):
                    aggl[pl.ds(i + u * 16, 16)] = z
                    cntl[pl.ds(i + u * 16, 16)] = z

            for c in cps:
                c.wait()

            phl = jnp.zeros((16,), jnp.float32)
            phr = jnp.zeros((16,), jnp.float32)
            for u in range(4):
                phl = phl + phl_v[pl.ds(u * 16, 16)]
                phr = phr + phr_v[pl.ds(u * 16, 16)]
            wv = plsc.load_gather(xwbuf, [bv]) * 34
            ew0 = plsc.load_gather(ewbuf, [wv + iota])
            ew1 = plsc.load_gather(ewbuf, [wv + (iota + 16)])
            ew2 = plsc.load_gather(
                ewbuf, [jnp.minimum(wv + (iota + 32), 237)])
            xfg = plsc.load_gather(
                xfbuf, [jnp.clip(iota + (2 * b - 2), 0, 127)])
            chunk2 = jnp.where(iota < 2, ew2,
                               jnp.where(iota < 4, xfg, 0.0))
            cl = jnp.sum(ew0 * wlA + ew1 * wlB + chunk2 * wlC + phl)
            crbl = blv + jnp.sum(ew0 * wrA + ew1 * wrB + chunk2 * wrC + phr)

            for c in range(4):
                toff = c * 16
                v = (osrc_v[pl.ds(toff, 16)] * wl48
                     + xd_v[pl.ds(toff, 16)] * wl49)
                valid = (iota + toff) < (_TRAJ - 1)
                dstv = plsc.load_gather(
                    sb, [jnp.minimum(iota + (toff + 1), _TRAJ - 1)])
                for j in range(16):
                    m = valid & masks[j]
                    plsc.addupdate_scatter(aggl, [dstv], v, mask=m)
                    plsc.addupdate_scatter(cntl, [dstv], ones16, mask=m)

            @pl.loop(0, _NN, step=80)
            def _(i):
                for u in range(5):
                    sl = pl.ds(i + u * 16, 16)
                    cn = cntl[sl]
                    mean = aggl[sl] / jnp.maximum(cn, 1.0)
                    aggl[sl] = (mean + jnp.where(cn >= 0.5, cl, 0.0)
                                + crbl)

            out_copies.append(pltpu.async_copy(
                aggl, sage_hbm.at[pl.ds(boff, _NN)], sem_out))

        for c in out_copies:
            c.wait()

    return sck(stops, out_flat, phi_flat, x_dist, pack, x_week, x_feat_flat,
               emb_week_flat)


def _tc_b(out2, sage2, xd2, Wr):
    def body(out_ref, sage_ref, xd_ref, wr_ref, o_ref):
        wr = wr_ref[0, :]
        wr48 = wr[48:49]
        wr49 = wr[49:50]
        rowi = lax.broadcasted_iota(jnp.int32, (_RPB, 128), 0)
        lanei = lax.broadcasted_iota(jnp.int32, (_RPB, 128), 1)
        vmask = (rowi * 128 + lanei) < _NN
        xdw = xd_ref[...] * wr49
        gs = []
        for r in range(8):
            sl = pl.ds(r * _RPB, _RPB)
            g = sage_ref[sl, :] + out_ref[sl, :] * wr48 + xdw
            gs.append(jnp.where(vmask, g, -1e30))
        gms = [jnp.max(g) for g in gs]
        es = [jnp.exp(g - gm) for g, gm in zip(gs, gms)]
        lses = [gm + jnp.log(jnp.sum(e)) for gm, e in zip(gms, es)]
        for r in range(8):
            o_ref[pl.ds(r * _RPB, _RPB), :] = gs[r] - lses[r]

    return pl.pallas_call(
        body,
        grid=(8,),
        in_specs=[
            pl.BlockSpec((8 * _RPB, 128), lambda i: (i, 0)),
            pl.BlockSpec((8 * _RPB, 128), lambda i: (i, 0)),
            pl.BlockSpec((_RPB, 128), lambda i: (0, 0)),
            pl.BlockSpec((1, 50), lambda i: (0, 0)),
        ],
        out_specs=pl.BlockSpec((8 * _RPB, 128), lambda i: (i, 0)),
        out_shape=jax.ShapeDtypeStruct((_B * _RPB, 128), jnp.float32),
    )(out2, sage2, xd2, Wr)


def kernel(stops, x, x_dist, x_features, x_week, x_mask, emb_week, emb_stop,
           fc1_W, fc1_b, Wl, bl, Wr):
    f32 = jnp.float32
    stops32 = stops.astype(jnp.int32)
    x_week32 = x_week.astype(jnp.int32)
    z12 = jnp.zeros((12,), f32)
    pack = jnp.concatenate([
        Wl[0, 48:50], bl, jnp.zeros((13,), f32),
        Wl[0, 0:16], Wl[0, 16:32], Wl[0, 32:36], z12, jnp.zeros((16,), f32),
        Wr[0, 0:16], Wr[0, 16:32], Wr[0, 32:36], z12, jnp.zeros((16,), f32),
    ])
    xf_flat = x_features.reshape(-1).astype(f32)
    ew_flat = jnp.concatenate([emb_week.reshape(-1), jnp.zeros((2,), f32)])
    xd2 = jnp.concatenate(
        [x_dist, jnp.zeros((_NP - _NN,), f32)]).reshape(_RPB, 128)

    out_flat, phi_flat = _tc_a(x, fc1_W, fc1_b, emb_stop, Wl, Wr)
    sage_flat = _sc_kernel(stops32, out_flat, phi_flat, x_dist, pack,
                           x_week32, xf_flat, ew_flat)
    logp = _tc_b(out_flat.reshape(_B * _RPB, 128),
                 sage_flat.reshape(_B * _RPB, 128), xd2, Wr)
    logp = logp.reshape(_B, _NP)[:, :_NN]
    return jnp.where(x_mask.astype(bool), jnp.float32(-1e8), logp)
